# Initial kernel scaffold; baseline (speedup 1.0000x reference)
import functools

import jax
import jax.numpy as jnp
from jax import lax
from jax.experimental import pallas as pl
from jax.experimental.pallas import tpu as pltpu
from jax.experimental.pallas import tpu_sc as plsc

N = 10000
E = 320000
NC = 2
NS = 16
CHUNK = 80
ROWS_PER_TILE = N // NS
DEG_W = 16
BLK = 1000


def _mesh():
    return plsc.VectorSubcoreMesh(
        core_axis_name="c", subcore_axis_name="s", num_cores=NC, num_subcores=NS
    )


def _make_degree_kernel():
    edges_per_worker = E // (NC * NS)
    nchunks = edges_per_worker // CHUNK

    @functools.partial(
        pl.kernel,
        out_type=(
            jax.ShapeDtypeStruct((NC, N, DEG_W), jnp.float32),
            jax.ShapeDtypeStruct((NC, N, DEG_W), jnp.float32),
        ),
        mesh=_mesh(),
        scratch_types=[
            pltpu.VMEM((CHUNK,), jnp.int32),
            pltpu.VMEM((CHUNK,), jnp.int32),
            pltpu.VMEM((CHUNK, DEG_W), jnp.float32),
            pltpu.VMEM((ROWS_PER_TILE, DEG_W), jnp.float32),
            pltpu.VMEM_SHARED((N, DEG_W), jnp.float32),
            pltpu.VMEM_SHARED((N, DEG_W), jnp.float32),
        ],
    )
    def deg(edge_hbm, out_src, out_dst, src_v, dst_v, ones_v, zero_v, acc_s, acc_d):
        c = lax.axis_index("c")
        s = lax.axis_index("s")
        wid = s * NC + c

        def fill_ones(i, carry):
            ones_v[i, :] = jnp.full((16,), 1.0, jnp.float32)
            return carry

        lax.fori_loop(0, CHUNK, fill_ones, 0)

        def fill_zero(i, carry):
            zero_v[i, :] = jnp.zeros((16,), jnp.float32)
            return carry

        lax.fori_loop(0, ROWS_PER_TILE, fill_zero, 0)

        row0 = s * ROWS_PER_TILE
        pltpu.sync_copy(zero_v, acc_s.at[pl.ds(row0, ROWS_PER_TILE)])
        pltpu.sync_copy(zero_v, acc_d.at[pl.ds(row0, ROWS_PER_TILE)])
        plsc.subcore_barrier()

        base0 = wid * edges_per_worker

        def body(i, carry):
            base = base0 + i * CHUNK
            pltpu.sync_copy(edge_hbm.at[0, pl.ds(base, CHUNK)], src_v)
            pltpu.sync_copy(edge_hbm.at[1, pl.ds(base, CHUNK)], dst_v)
            pltpu.sync_copy(ones_v, acc_s.at[src_v], add=True)
            pltpu.sync_copy(ones_v, acc_d.at[dst_v], add=True)
            return carry

        lax.fori_loop(0, nchunks, body, 0)
        plsc.subcore_barrier()

        pltpu.sync_copy(
            acc_s.at[pl.ds(row0, ROWS_PER_TILE)],
            out_src.at[c, pl.ds(row0, ROWS_PER_TILE)],
        )
        pltpu.sync_copy(
            acc_d.at[pl.ds(row0, ROWS_PER_TILE)],
            out_dst.at[c, pl.ds(row0, ROWS_PER_TILE)],
        )

    return deg


def _make_agg_kernel(dh):
    edges_per_tile = E // NS
    nchunks = edges_per_tile // CHUNK
    zrows = 125
    vecs = dh // 16

    @functools.partial(
        pl.kernel,
        out_type=jax.ShapeDtypeStruct((NC, N, dh), jnp.float32),
        mesh=_mesh(),
        scratch_types=[
            pltpu.VMEM((CHUNK,), jnp.int32),
            pltpu.VMEM((CHUNK,), jnp.int32),
            pltpu.VMEM((CHUNK, dh), jnp.float32),
            pltpu.VMEM((zrows, dh), jnp.float32),
            pltpu.VMEM_SHARED((N, dh), jnp.float32),
            pltpu.SemaphoreType.DMA,
        ],
    )
    def agg(tbl0, tbl1, edge_hbm, out_hbm, src_v, dst_v, rows_v, zero_v, acc, sem):
        c = lax.axis_index("c")
        s = lax.axis_index("s")

        def fill_zero(k, carry):
            i = k // vecs
            j = k % vecs
            zero_v[i, pl.ds(j * 16, 16)] = jnp.zeros((16,), jnp.float32)
            return carry

        lax.fori_loop(0, zrows * vecs, fill_zero, 0)

        row0 = s * ROWS_PER_TILE
        for r in range(ROWS_PER_TILE // zrows):
            pltpu.sync_copy(zero_v, acc.at[pl.ds(row0 + r * zrows, zrows)])
        plsc.subcore_barrier()

        base0 = s * edges_per_tile

        def body(i, carry):
            base = base0 + i * CHUNK
            pltpu.sync_copy(edge_hbm.at[0, pl.ds(base, CHUNK)], src_v)
            pltpu.sync_copy(edge_hbm.at[1, pl.ds(base, CHUNK)], dst_v)

            @pl.when(c == 0)
            def _():
                pltpu.async_copy(tbl0.at[src_v], rows_v, sem).wait()

            @pl.when(c == 1)
            def _():
                pltpu.async_copy(tbl1.at[src_v], rows_v, sem).wait()

            pltpu.sync_copy(rows_v, acc.at[dst_v], add=True)
            return carry

        lax.fori_loop(0, nchunks, body, 0)
        plsc.subcore_barrier()

        pltpu.sync_copy(
            acc.at[pl.ds(row0, ROWS_PER_TILE)],
            out_hbm.at[c, pl.ds(row0, ROWS_PER_TILE)],
        )

    return agg


def _tc_norm(scnt, dcnt, feat):
    def body(s_ref, d_ref, f_ref, ta_ref, tb_ref, in_ref, on_ref):
        outd = s_ref[0, :, :1] + s_ref[1, :, :1]
        ind = d_ref[0, :, :1] + d_ref[1, :, :1]
        onorm = lax.rsqrt(jnp.maximum(outd, 1.0))
        inorm = lax.rsqrt(jnp.maximum(ind, 1.0))
        f = f_ref[...]
        ta_ref[...] = f[:, :64] * onorm
        tb_ref[...] = f[:, 64:] * onorm
        in_ref[...] = jnp.broadcast_to(inorm, (BLK, 8))
        on_ref[...] = jnp.broadcast_to(onorm, (BLK, 8))

    return pl.pallas_call(
        body,
        grid=(N // BLK,),
        in_specs=[
            pl.BlockSpec((NC, BLK, DEG_W), lambda i: (0, i, 0)),
            pl.BlockSpec((NC, BLK, DEG_W), lambda i: (0, i, 0)),
            pl.BlockSpec((BLK, 128), lambda i: (i, 0)),
        ],
        out_specs=[
            pl.BlockSpec((BLK, 64), lambda i: (i, 0)),
            pl.BlockSpec((BLK, 64), lambda i: (i, 0)),
            pl.BlockSpec((BLK, 8), lambda i: (i, 0)),
            pl.BlockSpec((BLK, 8), lambda i: (i, 0)),
        ],
        out_shape=[
            jax.ShapeDtypeStruct((N, 64), jnp.float32),
            jax.ShapeDtypeStruct((N, 64), jnp.float32),
            jax.ShapeDtypeStruct((N, 8), jnp.float32),
            jax.ShapeDtypeStruct((N, 8), jnp.float32),
        ],
    )(scnt, dcnt, feat)


def _tc_dense(aggr, inorm, onorm, W, b, dh):
    d_in = 2 * dh
    d_out = W.shape[1]

    def body(a_ref, in_ref, on_ref, w_ref, b_ref, oa_ref, ob_ref):
        h = jnp.concatenate([a_ref[0], a_ref[1]], axis=1) * in_ref[:, :1]
        z = jnp.dot(h, w_ref[...], preferred_element_type=jnp.float32) + b_ref[...]
        z = jnp.where(z >= 0.0, z, 0.01 * z)
        zs = z * on_ref[:, :1]
        oa_ref[...] = zs[:, : d_out // 2]
        ob_ref[...] = zs[:, d_out // 2 :]

    return pl.pallas_call(
        body,
        grid=(N // BLK,),
        in_specs=[
            pl.BlockSpec((NC, BLK, dh), lambda i: (0, i, 0)),
            pl.BlockSpec((BLK, 8), lambda i: (i, 0)),
            pl.BlockSpec((BLK, 8), lambda i: (i, 0)),
            pl.BlockSpec((d_in, d_out), lambda i: (0, 0)),
            pl.BlockSpec((1, d_out), lambda i: (0, 0)),
        ],
        out_specs=[
            pl.BlockSpec((BLK, d_out // 2), lambda i: (i, 0)),
            pl.BlockSpec((BLK, d_out // 2), lambda i: (i, 0)),
        ],
        out_shape=[
            jax.ShapeDtypeStruct((N, d_out // 2), jnp.float32),
            jax.ShapeDtypeStruct((N, d_out // 2), jnp.float32),
        ],
    )(aggr, inorm, onorm, W, b)


def _tc_final(aggr, inorm, W3, b3, Wm1, bm1, Wm2, bm2):
    dh = 128

    def body(a_ref, in_ref, w_ref, b_ref, w1_ref, b1_ref, w2_ref, b2_ref, o_ref):
        h = jnp.concatenate([a_ref[0], a_ref[1]], axis=1) * in_ref[:, :1]
        z = jnp.dot(h, w_ref[...], preferred_element_type=jnp.float32) + b_ref[...]
        z = jnp.where(z >= 0.0, z, 0.01 * z)
        m = jnp.dot(z, w1_ref[...], preferred_element_type=jnp.float32) + b1_ref[...]
        m = jnp.maximum(m, 0.0)
        o = jnp.dot(m, w2_ref[...], preferred_element_type=jnp.float32) + b2_ref[...]
        o_ref[...] = jnp.maximum(o, 0.0)

    return pl.pallas_call(
        body,
        grid=(N // BLK,),
        in_specs=[
            pl.BlockSpec((NC, BLK, dh), lambda i: (0, i, 0)),
            pl.BlockSpec((BLK, 8), lambda i: (i, 0)),
            pl.BlockSpec((256, 256), lambda i: (0, 0)),
            pl.BlockSpec((1, 256), lambda i: (0, 0)),
            pl.BlockSpec((256, 32), lambda i: (0, 0)),
            pl.BlockSpec((1, 32), lambda i: (0, 0)),
            pl.BlockSpec((32, 2), lambda i: (0, 0)),
            pl.BlockSpec((1, 2), lambda i: (0, 0)),
        ],
        out_specs=pl.BlockSpec((BLK, 2), lambda i: (i, 0)),
        out_shape=jax.ShapeDtypeStruct((N, 2), jnp.float32),
    )(aggr, inorm, W3, b3, Wm1, bm1, Wm2, bm2)


_degree_call = _make_degree_kernel()
_agg64_call = _make_agg_kernel(64)
_agg128_call = _make_agg_kernel(128)


@jax.jit
def kernel(features, edge_index, W1, b1, W2, b2, W3, b3, Wm1, bm1, Wm2, bm2):
    scnt, dcnt = _degree_call(edge_index)
    ta, tb, inorm, onorm = _tc_norm(scnt, dcnt, features)

    agg1 = _agg64_call(ta, tb, edge_index)
    ta, tb = _tc_dense(agg1, inorm, onorm, W1, b1.reshape(1, -1), 64)

    agg2 = _agg128_call(ta, tb, edge_index)
    ta, tb = _tc_dense(agg2, inorm, onorm, W2, b2.reshape(1, -1), 128)

    agg3 = _agg128_call(ta, tb, edge_index)
    return _tc_final(
        agg3, inorm, W3, b3.reshape(1, -1),
        Wm1, bm1.reshape(1, -1), Wm2, bm2.reshape(1, -1),
    )


# speedup vs baseline: 2.8248x; 2.8248x over previous
import functools

import jax
import jax.numpy as jnp
from jax import lax
from jax.experimental import pallas as pl
from jax.experimental.pallas import tpu as pltpu
from jax.experimental.pallas import tpu_sc as plsc

N = 10000
E = 320000
NC = 2
NS = 16
CHUNK = 80
N_PAD = 10240
ROWS_PER_TILE = N_PAD // NS
ZR = 16
BLK = 1000


def _mesh():
    return plsc.VectorSubcoreMesh(
        core_axis_name="c", subcore_axis_name="s", num_cores=NC, num_subcores=NS
    )


def _make_degree_kernel():
    edges_per_worker = E // (NC * NS)
    nchunks = edges_per_worker // CHUNK

    @functools.partial(
        pl.kernel,
        out_type=jax.ShapeDtypeStruct((NC, N_PAD, 128), jnp.float32),
        mesh=_mesh(),
        scratch_types=[
            pltpu.VMEM((CHUNK,), jnp.int32),
            pltpu.VMEM((CHUNK,), jnp.int32),
            pltpu.VMEM((CHUNK, 128), jnp.float32),
            pltpu.VMEM((CHUNK, 128), jnp.float32),
            pltpu.VMEM((ZR, 128), jnp.float32),
            pltpu.VMEM_SHARED((N_PAD, 128), jnp.float32),
        ],
    )
    def deg(src_hbm, dst_hbm, ones_s_hbm, ones_d_hbm, out, src_v, dst_v,
            ones_s, ones_d, zero_v, acc):
        c = lax.axis_index("c")
        s = lax.axis_index("s")
        wid = s * NC + c

        pltpu.sync_copy(ones_s_hbm, ones_s)
        pltpu.sync_copy(ones_d_hbm, ones_d)

        def fill_zero(k, carry):
            i = k // 8
            j = k % 8
            zero_v[i, pl.ds(j * 16, 16)] = jnp.zeros((16,), jnp.float32)
            return carry

        lax.fori_loop(0, ZR * 8, fill_zero, 0)

        row0 = s * ROWS_PER_TILE
        for r in range(ROWS_PER_TILE // ZR):
            pltpu.sync_copy(zero_v, acc.at[pl.ds(row0 + r * ZR, ZR)])
        plsc.subcore_barrier()

        base0 = wid * edges_per_worker

        def body(i, carry):
            base = base0 + i * CHUNK
            pltpu.sync_copy(src_hbm.at[pl.ds(base, CHUNK)], src_v)
            pltpu.sync_copy(dst_hbm.at[pl.ds(base, CHUNK)], dst_v)
            pltpu.sync_copy(ones_s, acc.at[src_v], add=True)
            pltpu.sync_copy(ones_d, acc.at[dst_v], add=True)
            return carry

        lax.fori_loop(0, nchunks, body, 0)
        plsc.subcore_barrier()

        for r in range(ROWS_PER_TILE // ZR):
            pltpu.sync_copy(
                acc.at[pl.ds(row0 + r * ZR, ZR)],
                out.at[c, pl.ds(row0 + r * ZR, ZR)],
            )

    return deg


def _make_agg_kernel():
    edges_per_worker = E // (NC * NS)
    nchunks = edges_per_worker // CHUNK

    @functools.partial(
        pl.kernel,
        out_type=jax.ShapeDtypeStruct((NC, N_PAD, 128), jnp.float32),
        mesh=_mesh(),
        scratch_types=[
            pltpu.VMEM((CHUNK,), jnp.int32),
            pltpu.VMEM((CHUNK,), jnp.int32),
            pltpu.VMEM((CHUNK, 128), jnp.float32),
            pltpu.VMEM((ZR, 128), jnp.float32),
            pltpu.VMEM_SHARED((N_PAD, 128), jnp.float32),
            pltpu.SemaphoreType.DMA,
        ],
    )
    def agg(tbl, src_hbm, dst_hbm, out_hbm, src_v, dst_v, rows_v, zero_v, acc, sem):
        c = lax.axis_index("c")
        s = lax.axis_index("s")
        wid = s * NC + c

        def fill_zero(k, carry):
            i = k // 8
            j = k % 8
            zero_v[i, pl.ds(j * 16, 16)] = jnp.zeros((16,), jnp.float32)
            return carry

        lax.fori_loop(0, ZR * 8, fill_zero, 0)

        row0 = s * ROWS_PER_TILE
        for r in range(ROWS_PER_TILE // ZR):
            pltpu.sync_copy(zero_v, acc.at[pl.ds(row0 + r * ZR, ZR)])
        plsc.subcore_barrier()

        base0 = wid * edges_per_worker

        def body(i, carry):
            base = base0 + i * CHUNK
            pltpu.sync_copy(src_hbm.at[pl.ds(base, CHUNK)], src_v)
            pltpu.sync_copy(dst_hbm.at[pl.ds(base, CHUNK)], dst_v)
            pltpu.async_copy(tbl.at[src_v], rows_v, sem).wait()
            pltpu.sync_copy(rows_v, acc.at[dst_v], add=True)
            return carry

        lax.fori_loop(0, nchunks, body, 0)
        plsc.subcore_barrier()

        for r in range(ROWS_PER_TILE // ZR):
            pltpu.sync_copy(
                acc.at[pl.ds(row0 + r * ZR, ZR)],
                out_hbm.at[c, pl.ds(row0 + r * ZR, ZR)],
            )

    return agg


def _tc_norm(cnt, feat):
    def body(c_ref, f_ref, t_ref, in_ref, on_ref):
        outd = c_ref[0, :, :1] + c_ref[1, :, :1]
        ind = c_ref[0, :, 64:65] + c_ref[1, :, 64:65]
        onorm = lax.rsqrt(jnp.maximum(outd, 1.0))
        inorm = lax.rsqrt(jnp.maximum(ind, 1.0))
        t_ref[...] = f_ref[...] * onorm
        in_ref[...] = jnp.broadcast_to(inorm, (BLK, 8))
        on_ref[...] = jnp.broadcast_to(onorm, (BLK, 8))

    return pl.pallas_call(
        body,
        grid=(N // BLK,),
        in_specs=[
            pl.BlockSpec((NC, BLK, 128), lambda i: (0, i, 0)),
            pl.BlockSpec((BLK, 128), lambda i: (i, 0)),
        ],
        out_specs=[
            pl.BlockSpec((BLK, 128), lambda i: (i, 0)),
            pl.BlockSpec((BLK, 8), lambda i: (i, 0)),
            pl.BlockSpec((BLK, 8), lambda i: (i, 0)),
        ],
        out_shape=[
            jax.ShapeDtypeStruct((N, 128), jnp.float32),
            jax.ShapeDtypeStruct((N, 8), jnp.float32),
            jax.ShapeDtypeStruct((N, 8), jnp.float32),
        ],
    )(cnt, feat)


def _tc_dense1(agg, inorm, onorm, W, b):
    def body(a_ref, in_ref, on_ref, w_ref, b_ref, oa_ref, ob_ref):
        h = (a_ref[0] + a_ref[1]) * in_ref[:, :1]
        z = jnp.dot(h, w_ref[...], preferred_element_type=jnp.float32) + b_ref[...]
        z = jnp.where(z >= 0.0, z, 0.01 * z)
        zs = z * on_ref[:, :1]
        oa_ref[...] = zs[:, :128]
        ob_ref[...] = zs[:, 128:]

    return pl.pallas_call(
        body,
        grid=(N // BLK,),
        in_specs=[
            pl.BlockSpec((NC, BLK, 128), lambda i: (0, i, 0)),
            pl.BlockSpec((BLK, 8), lambda i: (i, 0)),
            pl.BlockSpec((BLK, 8), lambda i: (i, 0)),
            pl.BlockSpec((128, 256), lambda i: (0, 0)),
            pl.BlockSpec((1, 256), lambda i: (0, 0)),
        ],
        out_specs=[
            pl.BlockSpec((BLK, 128), lambda i: (i, 0)),
            pl.BlockSpec((BLK, 128), lambda i: (i, 0)),
        ],
        out_shape=[
            jax.ShapeDtypeStruct((N, 128), jnp.float32),
            jax.ShapeDtypeStruct((N, 128), jnp.float32),
        ],
    )(agg, inorm, onorm, W, b)


def _tc_dense2(aggA, aggB, inorm, onorm, W, b):
    def body(a_ref, b2_ref, in_ref, on_ref, w_ref, bias_ref, oa_ref, ob_ref):
        h = jnp.concatenate(
            [a_ref[0] + a_ref[1], b2_ref[0] + b2_ref[1]], axis=1
        ) * in_ref[:, :1]
        z = jnp.dot(h, w_ref[...], preferred_element_type=jnp.float32) + bias_ref[...]
        z = jnp.where(z >= 0.0, z, 0.01 * z)
        zs = z * on_ref[:, :1]
        oa_ref[...] = zs[:, :128]
        ob_ref[...] = zs[:, 128:]

    return pl.pallas_call(
        body,
        grid=(N // BLK,),
        in_specs=[
            pl.BlockSpec((NC, BLK, 128), lambda i: (0, i, 0)),
            pl.BlockSpec((NC, BLK, 128), lambda i: (0, i, 0)),
            pl.BlockSpec((BLK, 8), lambda i: (i, 0)),
            pl.BlockSpec((BLK, 8), lambda i: (i, 0)),
            pl.BlockSpec((256, 256), lambda i: (0, 0)),
            pl.BlockSpec((1, 256), lambda i: (0, 0)),
        ],
        out_specs=[
            pl.BlockSpec((BLK, 128), lambda i: (i, 0)),
            pl.BlockSpec((BLK, 128), lambda i: (i, 0)),
        ],
        out_shape=[
            jax.ShapeDtypeStruct((N, 128), jnp.float32),
            jax.ShapeDtypeStruct((N, 128), jnp.float32),
        ],
    )(aggA, aggB, inorm, onorm, W, b)


def _tc_final(aggA, aggB, inorm, W3, b3, Wm1, bm1, Wm2, bm2):
    def body(a_ref, b2_ref, in_ref, w_ref, bias_ref, w1_ref, b1_ref,
             w2_ref, b2b_ref, o_ref):
        h = jnp.concatenate(
            [a_ref[0] + a_ref[1], b2_ref[0] + b2_ref[1]], axis=1
        ) * in_ref[:, :1]
        z = jnp.dot(h, w_ref[...], preferred_element_type=jnp.float32) + bias_ref[...]
        z = jnp.where(z >= 0.0, z, 0.01 * z)
        m = jnp.dot(z, w1_ref[...], preferred_element_type=jnp.float32) + b1_ref[...]
        m = jnp.maximum(m, 0.0)
        o = jnp.dot(m, w2_ref[...], preferred_element_type=jnp.float32) + b2b_ref[...]
        o_ref[...] = jnp.maximum(o, 0.0)

    return pl.pallas_call(
        body,
        grid=(N // BLK,),
        in_specs=[
            pl.BlockSpec((NC, BLK, 128), lambda i: (0, i, 0)),
            pl.BlockSpec((NC, BLK, 128), lambda i: (0, i, 0)),
            pl.BlockSpec((BLK, 8), lambda i: (i, 0)),
            pl.BlockSpec((256, 256), lambda i: (0, 0)),
            pl.BlockSpec((1, 256), lambda i: (0, 0)),
            pl.BlockSpec((256, 32), lambda i: (0, 0)),
            pl.BlockSpec((1, 32), lambda i: (0, 0)),
            pl.BlockSpec((32, 2), lambda i: (0, 0)),
            pl.BlockSpec((1, 2), lambda i: (0, 0)),
        ],
        out_specs=pl.BlockSpec((BLK, 2), lambda i: (i, 0)),
        out_shape=jax.ShapeDtypeStruct((N, 2), jnp.float32),
    )(aggA, aggB, inorm, W3, b3, Wm1, bm1, Wm2, bm2)


_degree_call = _make_degree_kernel()
_agg_call = _make_agg_kernel()

_ONES_S = jnp.concatenate(
    [jnp.ones((CHUNK, 64), jnp.float32), jnp.zeros((CHUNK, 64), jnp.float32)], axis=1
)
_ONES_D = jnp.concatenate(
    [jnp.zeros((CHUNK, 64), jnp.float32), jnp.ones((CHUNK, 64), jnp.float32)], axis=1
)


@jax.jit
def kernel(features, edge_index, W1, b1, W2, b2, W3, b3, Wm1, bm1, Wm2, bm2):
    src_ids = edge_index[0]
    dst_ids = edge_index[1]
    cnt = _degree_call(src_ids, dst_ids, _ONES_S, _ONES_D)
    t1, inorm, onorm = _tc_norm(cnt, features)

    agg1 = _agg_call(t1, src_ids, dst_ids)
    ta, tb = _tc_dense1(agg1, inorm, onorm, W1, b1.reshape(1, -1))

    agg2a = _agg_call(ta, src_ids, dst_ids)
    agg2b = _agg_call(tb, src_ids, dst_ids)
    ta, tb = _tc_dense2(agg2a, agg2b, inorm, onorm, W2, b2.reshape(1, -1))

    agg3a = _agg_call(ta, src_ids, dst_ids)
    agg3b = _agg_call(tb, src_ids, dst_ids)
    return _tc_final(
        agg3a, agg3b, inorm, W3, b3.reshape(1, -1),
        Wm1, bm1.reshape(1, -1), Wm2, bm2.reshape(1, -1),
    )


# speedup vs baseline: 5.3156x; 1.8818x over previous
import functools

import jax
import jax.numpy as jnp
from jax import lax
from jax.experimental import pallas as pl
from jax.experimental.pallas import tpu as pltpu
from jax.experimental.pallas import tpu_sc as plsc

N = 10000
E = 320000
NC = 2
NS = 16
CHUNK = 80
N_PAD = 10240
ROWS_PER_TILE = N_PAD // NS
ZR = 16
BLK = 1000


def _mesh():
    return plsc.VectorSubcoreMesh(
        core_axis_name="c", subcore_axis_name="s", num_cores=NC, num_subcores=NS
    )


def _make_degree_kernel():
    edges_per_worker = E // (NC * NS)
    nchunks = edges_per_worker // CHUNK

    @functools.partial(
        pl.kernel,
        out_type=jax.ShapeDtypeStruct((NC, N_PAD, 128), jnp.float32),
        mesh=_mesh(),
        scratch_types=[
            pltpu.VMEM((CHUNK,), jnp.int32),
            pltpu.VMEM((CHUNK,), jnp.int32),
            pltpu.VMEM((CHUNK, 128), jnp.float32),
            pltpu.VMEM((CHUNK, 128), jnp.float32),
            pltpu.VMEM((ZR, 128), jnp.float32),
            pltpu.VMEM_SHARED((N_PAD, 128), jnp.float32),
            pltpu.SemaphoreType.DMA,
        ],
    )
    def deg(src_hbm, dst_hbm, ones_s_hbm, ones_d_hbm, out, src_v, dst_v,
            ones_s, ones_d, zero_v, acc, zsem):
        c = lax.axis_index("c")
        s = lax.axis_index("s")
        wid = s * NC + c
        base0 = wid * edges_per_worker
        pltpu.sync_copy(ones_s_hbm, ones_s)
        pltpu.sync_copy(ones_d_hbm, ones_d)

        def fill_zero(k, carry):
            i = k // 8
            j = k % 8
            zero_v[i, pl.ds(j * 16, 16)] = jnp.zeros((16,), jnp.float32)
            return carry

        lax.fori_loop(0, ZR * 8, fill_zero, 0)

        row0 = s * ROWS_PER_TILE
        nz = ROWS_PER_TILE // ZR
        for g in range(nz // 8):
            ds = [
                pltpu.async_copy(
                    zero_v, acc.at[pl.ds(row0 + (g * 8 + r) * ZR, ZR)], zsem
                )
                for r in range(8)
            ]
            for d in ds:
                d.wait()
        plsc.subcore_barrier()

        def body(i, carry):
            base = base0 + i * CHUNK
            pltpu.sync_copy(src_hbm.at[pl.ds(base, CHUNK)], src_v)
            pltpu.sync_copy(dst_hbm.at[pl.ds(base, CHUNK)], dst_v)
            pltpu.sync_copy(ones_s, acc.at[src_v], add=True)
            pltpu.sync_copy(ones_d, acc.at[dst_v], add=True)
            return carry

        lax.fori_loop(0, nchunks, body, 0)
        plsc.subcore_barrier()

        for g in range(nz // 8):
            ds = [
                pltpu.async_copy(
                    acc.at[pl.ds(row0 + (g * 8 + r) * ZR, ZR)],
                    out.at[c, pl.ds(row0 + (g * 8 + r) * ZR, ZR)],
                    zsem,
                )
                for r in range(8)
            ]
            for d in ds:
                d.wait()

    return deg


def _make_agg_kernel():
    edges_per_worker = E // (NC * NS)
    nchunks = edges_per_worker // CHUNK

    @functools.partial(
        pl.kernel,
        out_type=jax.ShapeDtypeStruct((NC, N_PAD, 128), jnp.float32),
        mesh=_mesh(),
        scratch_types=[
            pltpu.VMEM((E // CHUNK // (NC * NS), CHUNK), jnp.int32),
            pltpu.VMEM((CHUNK,), jnp.int32),
            pltpu.VMEM((CHUNK,), jnp.int32),
            pltpu.VMEM((CHUNK, 128), jnp.float32),
            pltpu.VMEM((CHUNK, 128), jnp.float32),
            pltpu.VMEM((ZR, 128), jnp.float32),
            pltpu.VMEM_SHARED((N_PAD, 128), jnp.float32),
            pltpu.SemaphoreType.DMA,
            pltpu.SemaphoreType.DMA,
            pltpu.SemaphoreType.DMA,
            pltpu.SemaphoreType.DMA,
            pltpu.SemaphoreType.DMA,
        ],
    )
    def agg(tbl, src_hbm, dst2_hbm, out_hbm, dst_i, sv0, sv1, rv0, rv1,
            zero_v, acc, gsem0, gsem1, asem0, asem1, zsem):
        c = lax.axis_index("c")
        s = lax.axis_index("s")
        wid = s * NC + c
        base0 = wid * edges_per_worker
        pltpu.sync_copy(dst2_hbm.at[wid], dst_i)

        def fill_zero(k, carry):
            i = k // 8
            j = k % 8
            zero_v[i, pl.ds(j * 16, 16)] = jnp.zeros((16,), jnp.float32)
            return carry

        lax.fori_loop(0, ZR * 8, fill_zero, 0)

        row0 = s * ROWS_PER_TILE
        nz = ROWS_PER_TILE // ZR
        for g in range(nz // 8):
            ds = [
                pltpu.async_copy(
                    zero_v, acc.at[pl.ds(row0 + (g * 8 + r) * ZR, ZR)], zsem
                )
                for r in range(8)
            ]
            for d in ds:
                d.wait()
        plsc.subcore_barrier()

        pltpu.sync_copy(src_hbm.at[pl.ds(base0, CHUNK)], sv0)
        pltpu.async_copy(tbl.at[sv0], rv0, gsem0)
        pltpu.sync_copy(src_hbm.at[pl.ds(base0 + CHUNK, CHUNK)], sv1)
        pltpu.async_copy(tbl.at[sv1], rv1, gsem1)

        def body(p, carry):
            i0 = 2 * p
            pltpu.make_async_copy(tbl.at[sv0], rv0, gsem0).wait()
            pltpu.async_copy(rv0, acc.at[dst_i.at[i0]], asem0, add=True)
            pltpu.make_async_copy(tbl.at[sv1], rv1, gsem1).wait()
            pltpu.async_copy(rv1, acc.at[dst_i.at[i0 + 1]], asem1, add=True)
            pltpu.make_async_copy(rv0, acc.at[dst_i.at[i0]], asem0).wait()
            pltpu.sync_copy(src_hbm.at[pl.ds(base0 + (i0 + 2) * CHUNK, CHUNK)], sv0)
            pltpu.async_copy(tbl.at[sv0], rv0, gsem0)
            pltpu.make_async_copy(rv1, acc.at[dst_i.at[i0 + 1]], asem1).wait()
            pltpu.sync_copy(src_hbm.at[pl.ds(base0 + (i0 + 3) * CHUNK, CHUNK)], sv1)
            pltpu.async_copy(tbl.at[sv1], rv1, gsem1)
            return carry

        npairs = (nchunks - 1) // 2 - 1
        lax.fori_loop(0, npairs, body, 0)

        i0 = 2 * npairs
        pltpu.make_async_copy(tbl.at[sv0], rv0, gsem0).wait()
        pltpu.async_copy(rv0, acc.at[dst_i.at[i0]], asem0, add=True)
        pltpu.make_async_copy(tbl.at[sv1], rv1, gsem1).wait()
        pltpu.async_copy(rv1, acc.at[dst_i.at[i0 + 1]], asem1, add=True)
        pltpu.make_async_copy(rv0, acc.at[dst_i.at[i0]], asem0).wait()
        pltpu.sync_copy(src_hbm.at[pl.ds(base0 + (i0 + 2) * CHUNK, CHUNK)], sv0)
        pltpu.async_copy(tbl.at[sv0], rv0, gsem0)
        pltpu.make_async_copy(tbl.at[sv0], rv0, gsem0).wait()
        pltpu.async_copy(rv0, acc.at[dst_i.at[i0 + 2]], asem0, add=True)
        pltpu.make_async_copy(rv0, acc.at[dst_i.at[i0 + 2]], asem0).wait()
        pltpu.make_async_copy(rv1, acc.at[dst_i.at[i0 + 1]], asem1).wait()
        plsc.subcore_barrier()

        for g in range(nz // 8):
            ds = [
                pltpu.async_copy(
                    acc.at[pl.ds(row0 + (g * 8 + r) * ZR, ZR)],
                    out_hbm.at[c, pl.ds(row0 + (g * 8 + r) * ZR, ZR)],
                    zsem,
                )
                for r in range(8)
            ]
            for d in ds:
                d.wait()

    return agg


def _tc_norm(cnt, feat):
    def body(c_ref, f_ref, t_ref, in_ref, on_ref):
        outd = c_ref[0, :, :1] + c_ref[1, :, :1]
        ind = c_ref[0, :, 64:65] + c_ref[1, :, 64:65]
        onorm = lax.rsqrt(jnp.maximum(outd, 1.0))
        inorm = lax.rsqrt(jnp.maximum(ind, 1.0))
        t_ref[...] = f_ref[...] * onorm
        in_ref[...] = jnp.broadcast_to(inorm, (BLK, 8))
        on_ref[...] = jnp.broadcast_to(onorm, (BLK, 8))

    return pl.pallas_call(
        body,
        grid=(N // BLK,),
        in_specs=[
            pl.BlockSpec((NC, BLK, 128), lambda i: (0, i, 0)),
            pl.BlockSpec((BLK, 128), lambda i: (i, 0)),
        ],
        out_specs=[
            pl.BlockSpec((BLK, 128), lambda i: (i, 0)),
            pl.BlockSpec((BLK, 8), lambda i: (i, 0)),
            pl.BlockSpec((BLK, 8), lambda i: (i, 0)),
        ],
        out_shape=[
            jax.ShapeDtypeStruct((N, 128), jnp.float32),
            jax.ShapeDtypeStruct((N, 8), jnp.float32),
            jax.ShapeDtypeStruct((N, 8), jnp.float32),
        ],
    )(cnt, feat)


def _tc_dense1(agg, inorm, onorm, W, b):
    def body(a_ref, in_ref, on_ref, w_ref, b_ref, oa_ref, ob_ref):
        h = (a_ref[0] + a_ref[1]) * in_ref[:, :1]
        z = jnp.dot(h, w_ref[...], preferred_element_type=jnp.float32) + b_ref[...]
        z = jnp.where(z >= 0.0, z, 0.01 * z)
        zs = z * on_ref[:, :1]
        oa_ref[...] = zs[:, :128]
        ob_ref[...] = zs[:, 128:]

    return pl.pallas_call(
        body,
        grid=(N // BLK,),
        in_specs=[
            pl.BlockSpec((NC, BLK, 128), lambda i: (0, i, 0)),
            pl.BlockSpec((BLK, 8), lambda i: (i, 0)),
            pl.BlockSpec((BLK, 8), lambda i: (i, 0)),
            pl.BlockSpec((128, 256), lambda i: (0, 0)),
            pl.BlockSpec((1, 256), lambda i: (0, 0)),
        ],
        out_specs=[
            pl.BlockSpec((BLK, 128), lambda i: (i, 0)),
            pl.BlockSpec((BLK, 128), lambda i: (i, 0)),
        ],
        out_shape=[
            jax.ShapeDtypeStruct((N, 128), jnp.float32),
            jax.ShapeDtypeStruct((N, 128), jnp.float32),
        ],
    )(agg, inorm, onorm, W, b)


def _tc_dense2(aggA, aggB, inorm, onorm, W, b):
    def body(a_ref, b2_ref, in_ref, on_ref, w_ref, bias_ref, oa_ref, ob_ref):
        h = jnp.concatenate(
            [a_ref[0] + a_ref[1], b2_ref[0] + b2_ref[1]], axis=1
        ) * in_ref[:, :1]
        z = jnp.dot(h, w_ref[...], preferred_element_type=jnp.float32) + bias_ref[...]
        z = jnp.where(z >= 0.0, z, 0.01 * z)
        zs = z * on_ref[:, :1]
        oa_ref[...] = zs[:, :128]
        ob_ref[...] = zs[:, 128:]

    return pl.pallas_call(
        body,
        grid=(N // BLK,),
        in_specs=[
            pl.BlockSpec((NC, BLK, 128), lambda i: (0, i, 0)),
            pl.BlockSpec((NC, BLK, 128), lambda i: (0, i, 0)),
            pl.BlockSpec((BLK, 8), lambda i: (i, 0)),
            pl.BlockSpec((BLK, 8), lambda i: (i, 0)),
            pl.BlockSpec((256, 256), lambda i: (0, 0)),
            pl.BlockSpec((1, 256), lambda i: (0, 0)),
        ],
        out_specs=[
            pl.BlockSpec((BLK, 128), lambda i: (i, 0)),
            pl.BlockSpec((BLK, 128), lambda i: (i, 0)),
        ],
        out_shape=[
            jax.ShapeDtypeStruct((N, 128), jnp.float32),
            jax.ShapeDtypeStruct((N, 128), jnp.float32),
        ],
    )(aggA, aggB, inorm, onorm, W, b)


def _tc_final(aggA, aggB, inorm, W3, b3, Wm1, bm1, Wm2, bm2):
    def body(a_ref, b2_ref, in_ref, w_ref, bias_ref, w1_ref, b1_ref,
             w2_ref, b2b_ref, o_ref):
        h = jnp.concatenate(
            [a_ref[0] + a_ref[1], b2_ref[0] + b2_ref[1]], axis=1
        ) * in_ref[:, :1]
        z = jnp.dot(h, w_ref[...], preferred_element_type=jnp.float32) + bias_ref[...]
        z = jnp.where(z >= 0.0, z, 0.01 * z)
        m = jnp.dot(z, w1_ref[...], preferred_element_type=jnp.float32) + b1_ref[...]
        m = jnp.maximum(m, 0.0)
        o = jnp.dot(m, w2_ref[...], preferred_element_type=jnp.float32) + b2b_ref[...]
        o_ref[...] = jnp.maximum(o, 0.0)

    return pl.pallas_call(
        body,
        grid=(N // BLK,),
        in_specs=[
            pl.BlockSpec((NC, BLK, 128), lambda i: (0, i, 0)),
            pl.BlockSpec((NC, BLK, 128), lambda i: (0, i, 0)),
            pl.BlockSpec((BLK, 8), lambda i: (i, 0)),
            pl.BlockSpec((256, 256), lambda i: (0, 0)),
            pl.BlockSpec((1, 256), lambda i: (0, 0)),
            pl.BlockSpec((256, 32), lambda i: (0, 0)),
            pl.BlockSpec((1, 32), lambda i: (0, 0)),
            pl.BlockSpec((32, 2), lambda i: (0, 0)),
            pl.BlockSpec((1, 2), lambda i: (0, 0)),
        ],
        out_specs=pl.BlockSpec((BLK, 2), lambda i: (i, 0)),
        out_shape=jax.ShapeDtypeStruct((N, 2), jnp.float32),
    )(aggA, aggB, inorm, W3, b3, Wm1, bm1, Wm2, bm2)


_degree_call = _make_degree_kernel()
_agg_call = _make_agg_kernel()

_ONES_S = jnp.concatenate(
    [jnp.ones((CHUNK, 64), jnp.float32), jnp.zeros((CHUNK, 64), jnp.float32)], axis=1
)
_ONES_D = jnp.concatenate(
    [jnp.zeros((CHUNK, 64), jnp.float32), jnp.ones((CHUNK, 64), jnp.float32)], axis=1
)


@jax.jit
def kernel(features, edge_index, W1, b1, W2, b2, W3, b3, Wm1, bm1, Wm2, bm2):
    nw = NC * NS
    src1 = edge_index[0]
    dst2 = edge_index[1].reshape(nw, E // CHUNK // nw, CHUNK)
    cnt = _degree_call(edge_index[0], edge_index[1], _ONES_S, _ONES_D)
    t1, inorm, onorm = _tc_norm(cnt, features)

    agg1 = _agg_call(t1, src1, dst2)
    ta, tb = _tc_dense1(agg1, inorm, onorm, W1, b1.reshape(1, -1))

    agg2a = _agg_call(ta, src1, dst2)
    agg2b = _agg_call(tb, src1, dst2)
    ta, tb = _tc_dense2(agg2a, agg2b, inorm, onorm, W2, b2.reshape(1, -1))

    agg3a = _agg_call(ta, src1, dst2)
    agg3b = _agg_call(tb, src1, dst2)
    return _tc_final(
        agg3a, agg3b, inorm, W3, b3.reshape(1, -1),
        Wm1, bm1.reshape(1, -1), Wm2, bm2.reshape(1, -1),
    )
